# baseline (speedup 1.0000x reference)
import functools

import jax
import jax.numpy as jnp
from jax import lax
from jax.experimental import pallas as pl
from jax.experimental.pallas import tpu as pltpu
from jax.experimental.pallas import tpu_sc as plsc

BATCH = 16384
D = 32
NW = 32

U_COLS = 1000001
O_COLS = 100001
U_TAIL = (U_COLS // 128) * 128
O_TAIL = (O_COLS // 128) * 128
U_BLOCKS = U_TAIL // 128
O_BLOCKS = O_TAIL // 128
U_BPW = -(-U_BLOCKS // NW)
O_BPW = -(-O_BLOCKS // NW)
CSZ = 1536
U_CHUNKS = -(-(U_BPW * 128) // CSZ)
O_CHUNKS = -(-(O_BPW * 128) // CSZ)
CAP = 4096
DUMP = 2 * BATCH

_mesh = plsc.VectorSubcoreMesh(core_axis_name="c", subcore_axis_name="s")


def _filter_pass(ids_v, pid_v, ppos_v, lo, hi):
    def row_body(r, n):
        for c in range(8):
            ids = ids_v[r, pl.ds(c * 16, 16)]
            posv = lax.iota(jnp.int32, 16) + (r * 128 + c * 16)
            m = (ids >= lo) & (ids < hi)
            nw = jnp.minimum(n, CAP - 16)
            plsc.store_compressed(pid_v.at[pl.ds(nw, 16)], ids, mask=m)
            plsc.store_compressed(ppos_v.at[pl.ds(nw, 16)], posv, mask=m)
            n = n + plsc.all_reduce_population_count(m)[0]
        return n

    return lax.fori_loop(0, 128, row_body, jnp.int32(0))


def _refilter(pid_v, ppos_v, n_pairs, c_lo, c_hi, cid_v, cpos_v):
    def vec_body(i, n):
        lane = lax.iota(jnp.int32, 16) + i * 16
        ids = pid_v[pl.ds(i * 16, 16)]
        posv = ppos_v[pl.ds(i * 16, 16)]
        m = (lane < n_pairs) & (ids >= c_lo) & (ids < c_hi)
        nw = jnp.minimum(n, CAP - 16)
        plsc.store_compressed(cid_v.at[pl.ds(nw, 16)], ids, mask=m)
        plsc.store_compressed(cpos_v.at[pl.ds(nw, 16)], posv, mask=m)
        return n + plsc.all_reduce_population_count(m)[0]

    return lax.fori_loop(0, (n_pairs + 15) // 16, vec_body, jnp.int32(0))


def _extract_scatter(src_v, transposed, base_col, pos_off,
                     cid_v, cpos_v, nc, rowbuf_v, wpos_v, scr_hbm, sem):
    def wave_body(wv, _):
        wbase = wv * 128

        def grp_body(g, _):
            lane = lax.iota(jnp.int32, 16) + (wbase + g * 16)
            mv = lane < nc
            idx = jnp.minimum(lane, CAP - 1)
            tvec = plsc.load_gather(cid_v, [idx]) - base_col
            pvec = plsc.load_gather(cpos_v, [idx]) + pos_off
            pvec = jnp.where(mv, pvec, -1)
            wpos_v[pl.ds(g * 16, 16)] = pvec
            rows = lax.iota(jnp.int32, 16) + g * 16
            for f in range(D):
                fvec = jnp.full((16,), f, jnp.int32)
                if transposed:
                    vals = plsc.load_gather(src_v, [fvec, tvec], mask=mv)
                else:
                    vals = plsc.load_gather(src_v, [tvec, fvec], mask=mv)
                plsc.store_scatter(rowbuf_v, [rows, fvec], vals, mask=mv)
            return 0

        lax.fori_loop(0, 8, grp_body, 0)
        pltpu.async_copy(
            rowbuf_v,
            scr_hbm.at[plsc.Indices(wpos_v, ignored_value=-1)],
            sem).wait()
        return 0

    lax.fori_loop(0, (nc + 127) // 128, wave_body, 0)


def _table_pipeline(ids2_hbm, tab_hbm, tail_hbm, scr_hbm, pos_off,
                    n_blocks, bpw, n_chunks, tail_lo, wid, rows_v, r128_v,
                    ids_v, pid_v, ppos_v, cid_v, cpos_v,
                    slab_v, rowbuf_v, wpos_v, tail_v, sem):
    lo_blk = jnp.minimum(wid * bpw, n_blocks)
    hi_blk = jnp.minimum(lo_blk + bpw, n_blocks)
    lo = lo_blk * 128
    hi = hi_blk * 128
    hi_f = jnp.where(wid == NW - 1, jnp.int32(1 << 30), hi)

    pltpu.async_copy(ids2_hbm.at[r128_v], ids_v, sem).wait()
    n_pairs = _filter_pass(ids_v, pid_v, ppos_v, lo, hi_f)

    def chunk_body(c, _):
        c_lo = lo + c * CSZ
        c_hi = jnp.minimum(c_lo + CSZ, hi)

        @pl.when(c_lo < c_hi)
        def _():
            dma_lo = pl.multiple_of(jnp.maximum(c_hi - CSZ, 0), 128)
            src = tab_hbm.at[:, pl.ds(dma_lo, CSZ)]
            pltpu.async_copy(src.at[rows_v], slab_v, sem).wait()
            nc = _refilter(pid_v, ppos_v, n_pairs, c_lo, c_hi, cid_v, cpos_v)
            _extract_scatter(slab_v, True, dma_lo, pos_off,
                             cid_v, cpos_v, nc, rowbuf_v, wpos_v,
                             scr_hbm, sem)

        return 0

    lax.fori_loop(0, n_chunks, chunk_body, 0)

    @pl.when(wid == NW - 1)
    def _():
        pltpu.async_copy(tail_hbm.at[r128_v], tail_v, sem).wait()
        nc = _refilter(pid_v, ppos_v, n_pairs, tail_lo, jnp.int32(1 << 30),
                       cid_v, cpos_v)
        _extract_scatter(tail_v, False, tail_lo, pos_off,
                         cid_v, cpos_v, nc, rowbuf_v, wpos_v, scr_hbm, sem)


@functools.partial(
    pl.kernel,
    out_type=jax.ShapeDtypeStruct((2 * BATCH + 1, 128), jnp.float32),
    mesh=_mesh,
    compiler_params=pltpu.CompilerParams(needs_layout_passes=False),
    scratch_types=[
        pltpu.VMEM((D,), jnp.int32),
        pltpu.VMEM((128,), jnp.int32),
        pltpu.VMEM((128, 128), jnp.int32),
        pltpu.VMEM((CAP,), jnp.int32),
        pltpu.VMEM((CAP,), jnp.int32),
        pltpu.VMEM((CAP,), jnp.int32),
        pltpu.VMEM((CAP,), jnp.int32),
        pltpu.VMEM((D, CSZ), jnp.float32),
        pltpu.VMEM((128, 128), jnp.float32),
        pltpu.VMEM((128,), jnp.int32),
        pltpu.VMEM((128, 128), jnp.float32),
        pltpu.SemaphoreType.DMA,
    ],
)
def _sweep(cid2_hbm, oid2_hbm, ut_hbm, ot_hbm, tailu_hbm, tailo_hbm, scr_hbm,
           rows_v, r128_v, ids_v, pid_v, ppos_v, cid_v, cpos_v, slab_v,
           rowbuf_v, wpos_v, tail_v, sem):
    wid = lax.axis_index("s") * 2 + lax.axis_index("c")
    for g in range(2):
        rows_v[pl.ds(g * 16, 16)] = lax.iota(jnp.int32, 16) + g * 16
    for g in range(8):
        r128_v[pl.ds(g * 16, 16)] = lax.iota(jnp.int32, 16) + g * 16

    _table_pipeline(cid2_hbm, ut_hbm, tailu_hbm, scr_hbm, 0,
                    U_BLOCKS, U_BPW, U_CHUNKS, U_TAIL, wid, rows_v, r128_v,
                    ids_v, pid_v, ppos_v, cid_v, cpos_v,
                    slab_v, rowbuf_v, wpos_v, tail_v, sem)

    _table_pipeline(oid2_hbm, ot_hbm, tailo_hbm, scr_hbm, BATCH,
                    O_BLOCKS, O_BPW, O_CHUNKS, O_TAIL, wid, rows_v, r128_v,
                    ids_v, pid_v, ppos_v, cid_v, cpos_v,
                    slab_v, rowbuf_v, wpos_v, tail_v, sem)


B_PER_W = BATCH // NW


@functools.partial(
    pl.kernel,
    out_type=jax.ShapeDtypeStruct((BATCH, 128), jnp.float32),
    mesh=_mesh,
    compiler_params=pltpu.CompilerParams(needs_layout_passes=False),
    scratch_types=[
        pltpu.VMEM((128, 128), jnp.float32),
        pltpu.VMEM((128, 128), jnp.float32),
        pltpu.VMEM((128, 128), jnp.float32),
        pltpu.VMEM((128,), jnp.int32),
        pltpu.SemaphoreType.DMA,
    ],
)
def _merge(scr_hbm, out_hbm, us_v, os_v, cat_v, widx_v, sem):
    wid = lax.axis_index("s") * 2 + lax.axis_index("c")
    base = wid * B_PER_W

    def blk_body(h, _):
        row0 = base + h * 128
        for g in range(8):
            widx_v[pl.ds(g * 16, 16)] = (lax.iota(jnp.int32, 16)
                                         + (row0 + g * 16))
        pltpu.async_copy(scr_hbm.at[widx_v], us_v, sem).wait()
        for g in range(8):
            widx_v[pl.ds(g * 16, 16)] = (lax.iota(jnp.int32, 16)
                                         + (BATCH + row0 + g * 16))
        pltpu.async_copy(scr_hbm.at[widx_v], os_v, sem).wait()

        def grp_body(g, _):
            rvec = lax.iota(jnp.int32, 16) + g * 16
            for f in range(D):
                fvec = jnp.full((16,), f, jnp.int32)
                uv = plsc.load_gather(us_v, [rvec, fvec])
                plsc.store_scatter(cat_v, [rvec, fvec], uv)
                ov = plsc.load_gather(os_v, [rvec, fvec])
                plsc.store_scatter(cat_v, [rvec, fvec + D], ov)
            return 0

        lax.fori_loop(0, 8, grp_body, 0)
        for g in range(8):
            widx_v[pl.ds(g * 16, 16)] = (lax.iota(jnp.int32, 16)
                                         + (row0 + g * 16))
        pltpu.async_copy(cat_v, out_hbm.at[widx_v], sem).wait()
        return 0

    lax.fori_loop(0, B_PER_W // 128, blk_body, 0)


def kernel(clientId, organization, user_table, org_table):
    cid2 = clientId.astype(jnp.int32).reshape(128, 128)
    oid2 = organization.astype(jnp.int32).reshape(128, 128)
    ut = user_table.T
    ot = org_table.T
    tail_u = jnp.pad(user_table[U_TAIL:],
                     ((0, 128 - (U_COLS - U_TAIL)), (0, 96)))
    tail_o = jnp.pad(org_table[O_TAIL:],
                     ((0, 128 - (O_COLS - O_TAIL)), (0, 96)))
    scr = _sweep(cid2, oid2, ut, ot, tail_u, tail_o)
    out = _merge(scr)
    return out[:, : 2 * D]


# speedup vs baseline: 1.0841x; 1.0841x over previous
import functools

import jax
import jax.numpy as jnp
from jax import lax
from jax.experimental import pallas as pl
from jax.experimental.pallas import tpu as pltpu
from jax.experimental.pallas import tpu_sc as plsc

BATCH = 16384
D = 32
NW = 32

U_COLS = 1000001
O_COLS = 100001
U_TAIL = (U_COLS // 128) * 128
O_TAIL = (O_COLS // 128) * 128
U_BLOCKS = U_TAIL // 128
O_BLOCKS = O_TAIL // 128
U_BPW = -(-U_BLOCKS // NW)
O_BPW = -(-O_BLOCKS // NW)
CSZ = 1536
U_CHUNKS = -(-(U_BPW * 128) // CSZ)
O_CHUNKS = -(-(O_BPW * 128) // CSZ)
CAP = 4096
DUMP = 2 * BATCH

_mesh = plsc.VectorSubcoreMesh(core_axis_name="c", subcore_axis_name="s")


def _filter_pass(ids_v, pid_v, ppos_v, lo, hi):
    def row_body(r, n):
        for c in range(8):
            ids = ids_v[r, pl.ds(c * 16, 16)]
            posv = lax.iota(jnp.int32, 16) + (r * 128 + c * 16)
            m = (ids >= lo) & (ids < hi)
            nw = jnp.minimum(n, CAP - 16)
            plsc.store_compressed(pid_v.at[pl.ds(nw, 16)], ids, mask=m)
            plsc.store_compressed(ppos_v.at[pl.ds(nw, 16)], posv, mask=m)
            n = n + plsc.all_reduce_population_count(m)[0]
        return n

    return lax.fori_loop(0, 128, row_body, jnp.int32(0))


def _refilter(pid_v, ppos_v, n_pairs, c_lo, c_hi, cid_v, cpos_v):
    def vec_body(i, n):
        lane = lax.iota(jnp.int32, 16) + i * 16
        ids = pid_v[pl.ds(i * 16, 16)]
        posv = ppos_v[pl.ds(i * 16, 16)]
        m = (lane < n_pairs) & (ids >= c_lo) & (ids < c_hi)
        nw = jnp.minimum(n, CAP - 16)
        plsc.store_compressed(cid_v.at[pl.ds(nw, 16)], ids, mask=m)
        plsc.store_compressed(cpos_v.at[pl.ds(nw, 16)], posv, mask=m)
        return n + plsc.all_reduce_population_count(m)[0]

    return lax.fori_loop(0, (n_pairs + 15) // 16, vec_body, jnp.int32(0))


def _extract_scatter(src_v, transposed, base_col, pos_off,
                     cid_v, cpos_v, nc, rowbuf_v, wpos_v, scr_hbm, sem):
    def wave_body(wv, _):
        wbase = wv * 128

        def grp_body(g, _):
            lane = lax.iota(jnp.int32, 16) + (wbase + g * 16)
            mv = lane < nc
            idx = jnp.minimum(lane, CAP - 1)
            tvec = plsc.load_gather(cid_v, [idx]) - base_col
            pvec = plsc.load_gather(cpos_v, [idx]) + pos_off
            pvec = jnp.where(mv, pvec, -1)
            wpos_v[pl.ds(g * 16, 16)] = pvec
            rows = lax.iota(jnp.int32, 16) + g * 16
            for f in range(D):
                fvec = jnp.full((16,), f, jnp.int32)
                if transposed:
                    vals = plsc.load_gather(src_v, [fvec, tvec], mask=mv)
                else:
                    vals = plsc.load_gather(src_v, [tvec, fvec], mask=mv)
                plsc.store_scatter(rowbuf_v, [rows, fvec], vals, mask=mv)
            return 0

        lax.fori_loop(0, 8, grp_body, 0)
        pltpu.async_copy(
            rowbuf_v,
            scr_hbm.at[plsc.Indices(wpos_v, ignored_value=-1)],
            sem).wait()
        return 0

    lax.fori_loop(0, (nc + 127) // 128, wave_body, 0)


def _table_pipeline(ids2_hbm, tab_hbm, tail_hbm, scr_hbm, pos_off,
                    n_blocks, bpw, n_chunks, tail_lo, wid, rows_v, r128_v,
                    ids_v, pid_v, ppos_v, cid_v, cpos_v,
                    slab_v, rowbuf_v, wpos_v, tail_v, sem):
    lo_blk = jnp.minimum(wid * bpw, n_blocks)
    hi_blk = jnp.minimum(lo_blk + bpw, n_blocks)
    lo = lo_blk * 128
    hi = hi_blk * 128
    hi_f = jnp.where(wid == NW - 1, jnp.int32(1 << 30), hi)

    pltpu.async_copy(ids2_hbm.at[r128_v], ids_v, sem).wait()
    n_pairs = _filter_pass(ids_v, pid_v, ppos_v, lo, hi_f)

    def chunk_body(c, _):
        c_lo = lo + c * CSZ
        c_hi = jnp.minimum(c_lo + CSZ, hi)

        @pl.when(c_lo < c_hi)
        def _():
            dma_lo = pl.multiple_of(jnp.maximum(c_hi - CSZ, 0), 128)
            src = tab_hbm.at[:, pl.ds(dma_lo, CSZ)]
            pltpu.async_copy(src.at[rows_v], slab_v, sem).wait()
            nc = _refilter(pid_v, ppos_v, n_pairs, c_lo, c_hi, cid_v, cpos_v)
            _extract_scatter(slab_v, True, dma_lo, pos_off,
                             cid_v, cpos_v, nc, rowbuf_v, wpos_v,
                             scr_hbm, sem)

        return 0

    lax.fori_loop(0, n_chunks, chunk_body, 0)

    @pl.when(wid == NW - 1)
    def _():
        pltpu.async_copy(tail_hbm.at[r128_v], tail_v, sem).wait()
        nc = _refilter(pid_v, ppos_v, n_pairs, tail_lo, jnp.int32(1 << 30),
                       cid_v, cpos_v)
        _extract_scatter(tail_v, False, tail_lo, pos_off,
                         cid_v, cpos_v, nc, rowbuf_v, wpos_v, scr_hbm, sem)


@functools.partial(
    pl.kernel,
    out_type=jax.ShapeDtypeStruct((2 * BATCH + 1, 128), jnp.float32),
    mesh=_mesh,
    compiler_params=pltpu.CompilerParams(needs_layout_passes=False),
    scratch_types=[
        pltpu.VMEM((D,), jnp.int32),
        pltpu.VMEM((128,), jnp.int32),
        pltpu.VMEM((128, 128), jnp.int32),
        pltpu.VMEM((CAP,), jnp.int32),
        pltpu.VMEM((CAP,), jnp.int32),
        pltpu.VMEM((CAP,), jnp.int32),
        pltpu.VMEM((CAP,), jnp.int32),
        pltpu.VMEM((D, CSZ), jnp.float32),
        pltpu.VMEM((128, 128), jnp.float32),
        pltpu.VMEM((128,), jnp.int32),
        pltpu.VMEM((128, 128), jnp.float32),
        pltpu.SemaphoreType.DMA,
    ],
)
def _sweep(cid2_hbm, oid2_hbm, ut_hbm, ot_hbm, tailu_hbm, tailo_hbm, scr_hbm,
           rows_v, r128_v, ids_v, pid_v, ppos_v, cid_v, cpos_v, slab_v,
           rowbuf_v, wpos_v, tail_v, sem):
    wid = lax.axis_index("s") * 2 + lax.axis_index("c")
    for g in range(2):
        rows_v[pl.ds(g * 16, 16)] = lax.iota(jnp.int32, 16) + g * 16
    for g in range(8):
        r128_v[pl.ds(g * 16, 16)] = lax.iota(jnp.int32, 16) + g * 16

    _table_pipeline(cid2_hbm, ut_hbm, tailu_hbm, scr_hbm, 0,
                    U_BLOCKS, U_BPW, U_CHUNKS, U_TAIL, wid, rows_v, r128_v,
                    ids_v, pid_v, ppos_v, cid_v, cpos_v,
                    slab_v, rowbuf_v, wpos_v, tail_v, sem)

    _table_pipeline(oid2_hbm, ot_hbm, tailo_hbm, scr_hbm, BATCH,
                    O_BLOCKS, O_BPW, O_CHUNKS, O_TAIL, wid, rows_v, r128_v,
                    ids_v, pid_v, ppos_v, cid_v, cpos_v,
                    slab_v, rowbuf_v, wpos_v, tail_v, sem)


B_PER_W = BATCH // NW


@functools.partial(
    pl.kernel,
    out_type=jax.ShapeDtypeStruct((BATCH, 128), jnp.float32),
    mesh=_mesh,
    compiler_params=pltpu.CompilerParams(needs_layout_passes=False),
    scratch_types=[
        pltpu.VMEM((128, 128), jnp.float32),
        pltpu.VMEM((128, 128), jnp.float32),
        pltpu.VMEM((128,), jnp.int32),
        pltpu.SemaphoreType.DMA,
    ],
)
def _merge(scr_hbm, out_hbm, us_v, os_v, widx_v, sem):
    wid = lax.axis_index("s") * 2 + lax.axis_index("c")
    base = wid * B_PER_W

    def blk_body(h, _):
        row0 = base + h * 128
        for g in range(8):
            widx_v[pl.ds(g * 16, 16)] = (lax.iota(jnp.int32, 16)
                                         + (row0 + g * 16))
        pltpu.async_copy(scr_hbm.at[widx_v], us_v, sem).wait()
        for g in range(8):
            widx_v[pl.ds(g * 16, 16)] = (lax.iota(jnp.int32, 16)
                                         + (BATCH + row0 + g * 16))
        pltpu.async_copy(scr_hbm.at[widx_v], os_v, sem).wait()

        def grp_body(g, _):
            rvec = lax.iota(jnp.int32, 16) + g * 16
            for f in range(D):
                fvec = jnp.full((16,), f, jnp.int32)
                ov = plsc.load_gather(os_v, [rvec, fvec])
                plsc.store_scatter(us_v, [rvec, fvec + D], ov)
            return 0

        lax.fori_loop(0, 8, grp_body, 0)
        for g in range(8):
            widx_v[pl.ds(g * 16, 16)] = (lax.iota(jnp.int32, 16)
                                         + (row0 + g * 16))
        pltpu.async_copy(us_v, out_hbm.at[widx_v], sem).wait()
        return 0

    lax.fori_loop(0, B_PER_W // 128, blk_body, 0)


def kernel(clientId, organization, user_table, org_table):
    cid2 = clientId.astype(jnp.int32).reshape(128, 128)
    oid2 = organization.astype(jnp.int32).reshape(128, 128)
    ut = user_table.T
    ot = org_table.T
    tail_u = jnp.pad(user_table[U_TAIL:],
                     ((0, 128 - (U_COLS - U_TAIL)), (0, 96)))
    tail_o = jnp.pad(org_table[O_TAIL:],
                     ((0, 128 - (O_COLS - O_TAIL)), (0, 96)))
    scr = _sweep(cid2, oid2, ut, ot, tail_u, tail_o)
    out = _merge(scr)
    return out[:, : 2 * D]


# speedup vs baseline: 1.1520x; 1.0626x over previous
import functools

import jax
import jax.numpy as jnp
from jax import lax
from jax.experimental import pallas as pl
from jax.experimental.pallas import tpu as pltpu
from jax.experimental.pallas import tpu_sc as plsc

BATCH = 16384
D = 32
NW = 32

U_COLS = 1000001
O_COLS = 100001
U_TAIL = (U_COLS // 128) * 128
O_TAIL = (O_COLS // 128) * 128
U_BLOCKS = U_TAIL // 128
O_BLOCKS = O_TAIL // 128
U_BPW = -(-U_BLOCKS // NW)
O_BPW = -(-O_BLOCKS // NW)
CSZ = 1536
U_CHUNKS = -(-(U_BPW * 128) // CSZ)
O_CHUNKS = -(-(O_BPW * 128) // CSZ)
CAP = 4096
DUMP = 2 * BATCH

_mesh = plsc.VectorSubcoreMesh(core_axis_name="c", subcore_axis_name="s")


def _filter_pass(ids_v, pid_v, ppos_v, lo, hi):
    def row_body(r, n):
        for c in range(8):
            ids = ids_v[r, pl.ds(c * 16, 16)]
            posv = lax.iota(jnp.int32, 16) + (r * 128 + c * 16)
            m = (ids >= lo) & (ids < hi)
            nw = jnp.minimum(n, CAP - 16)
            plsc.store_compressed(pid_v.at[pl.ds(nw, 16)], ids, mask=m)
            plsc.store_compressed(ppos_v.at[pl.ds(nw, 16)], posv, mask=m)
            n = n + plsc.all_reduce_population_count(m)[0]
        return n

    return lax.fori_loop(0, 128, row_body, jnp.int32(0))


def _refilter(pid_v, ppos_v, n_pairs, c_lo, c_hi, cid_v, cpos_v):
    def vec_body(i, n):
        lane = lax.iota(jnp.int32, 16) + i * 16
        ids = pid_v[pl.ds(i * 16, 16)]
        posv = ppos_v[pl.ds(i * 16, 16)]
        m = (lane < n_pairs) & (ids >= c_lo) & (ids < c_hi)
        nw = jnp.minimum(n, CAP - 16)
        plsc.store_compressed(cid_v.at[pl.ds(nw, 16)], ids, mask=m)
        plsc.store_compressed(cpos_v.at[pl.ds(nw, 16)], posv, mask=m)
        return n + plsc.all_reduce_population_count(m)[0]

    return lax.fori_loop(0, (n_pairs + 15) // 16, vec_body, jnp.int32(0))


def _extract_scatter(src_v, transposed, base_col, pos_off,
                     cid_v, cpos_v, nc, rowbuf_v, wpos_v, scr_hbm, sem):
    def wave_body(wv, _):
        wbase = wv * 128

        def grp_body(g, _):
            lane = lax.iota(jnp.int32, 16) + (wbase + g * 16)
            mv = lane < nc
            idx = jnp.minimum(lane, CAP - 1)
            tvec = plsc.load_gather(cid_v, [idx]) - base_col
            pvec = plsc.load_gather(cpos_v, [idx]) + pos_off
            pvec = jnp.where(mv, pvec, -1)
            wpos_v[pl.ds(g * 16, 16)] = pvec
            rows = lax.iota(jnp.int32, 16) + g * 16
            for f in range(D):
                fvec = jnp.full((16,), f, jnp.int32)
                if transposed:
                    vals = plsc.load_gather(src_v, [fvec, tvec], mask=mv)
                else:
                    vals = plsc.load_gather(src_v, [tvec, fvec], mask=mv)
                plsc.store_scatter(rowbuf_v, [rows, fvec], vals, mask=mv)
            return 0

        lax.fori_loop(0, 8, grp_body, 0)
        pltpu.async_copy(
            rowbuf_v,
            scr_hbm.at[plsc.Indices(wpos_v, ignored_value=-1)],
            sem).wait()
        return 0

    lax.fori_loop(0, (nc + 127) // 128, wave_body, 0)


def _table_pipeline(ids2_hbm, tab_hbm, tail_hbm, scr_hbm, pos_off,
                    n_blocks, bpw, n_chunks, tail_lo, wid, rows_v, r128_v,
                    ids_v, pid_v, ppos_v, cid_v, cpos_v,
                    slab_v, rowbuf_v, wpos_v, tail_v, sem):
    lo_blk = jnp.minimum(wid * bpw, n_blocks)
    hi_blk = jnp.minimum(lo_blk + bpw, n_blocks)
    lo = lo_blk * 128
    hi = hi_blk * 128
    hi_f = jnp.where(wid == NW - 1, jnp.int32(1 << 30), hi)

    pltpu.async_copy(ids2_hbm.at[r128_v], ids_v, sem).wait()
    n_pairs = _filter_pass(ids_v, pid_v, ppos_v, lo, hi_f)

    def chunk_body(c, _):
        c_lo = lo + c * CSZ
        c_hi = jnp.minimum(c_lo + CSZ, hi)

        @pl.when(c_lo < c_hi)
        def _():
            dma_lo = pl.multiple_of(jnp.maximum(c_hi - CSZ, 0), 128)
            src = tab_hbm.at[:, pl.ds(dma_lo, CSZ)]
            pltpu.async_copy(src.at[rows_v], slab_v, sem).wait()
            nc = _refilter(pid_v, ppos_v, n_pairs, c_lo, c_hi, cid_v, cpos_v)
            _extract_scatter(slab_v, True, dma_lo, pos_off,
                             cid_v, cpos_v, nc, rowbuf_v, wpos_v,
                             scr_hbm, sem)

        return 0

    lax.fori_loop(0, n_chunks, chunk_body, 0)

    @pl.when(wid == NW - 1)
    def _():
        pltpu.async_copy(tail_hbm.at[r128_v], tail_v, sem).wait()
        nc = _refilter(pid_v, ppos_v, n_pairs, tail_lo, jnp.int32(1 << 30),
                       cid_v, cpos_v)
        _extract_scatter(tail_v, False, tail_lo, pos_off,
                         cid_v, cpos_v, nc, rowbuf_v, wpos_v, scr_hbm, sem)


@functools.partial(
    pl.kernel,
    out_type=jax.ShapeDtypeStruct((2 * BATCH + 1, 128), jnp.float32),
    mesh=_mesh,
    compiler_params=pltpu.CompilerParams(needs_layout_passes=False),
    scratch_types=[
        pltpu.VMEM((D,), jnp.int32),
        pltpu.VMEM((128,), jnp.int32),
        pltpu.VMEM((128, 128), jnp.int32),
        pltpu.VMEM((CAP,), jnp.int32),
        pltpu.VMEM((CAP,), jnp.int32),
        pltpu.VMEM((CAP,), jnp.int32),
        pltpu.VMEM((CAP,), jnp.int32),
        pltpu.VMEM((D, CSZ), jnp.float32),
        pltpu.VMEM((128, 128), jnp.float32),
        pltpu.VMEM((128,), jnp.int32),
        pltpu.VMEM((128, 128), jnp.float32),
        pltpu.SemaphoreType.DMA,
    ],
)
def _sweep(cid2_hbm, oid2_hbm, ut_hbm, ot_hbm, tailu_hbm, tailo_hbm, scr_hbm,
           rows_v, r128_v, ids_v, pid_v, ppos_v, cid_v, cpos_v, slab_v,
           rowbuf_v, wpos_v, tail_v, sem):
    wid = lax.axis_index("s") * 2 + lax.axis_index("c")
    for g in range(2):
        rows_v[pl.ds(g * 16, 16)] = lax.iota(jnp.int32, 16) + g * 16
    for g in range(8):
        r128_v[pl.ds(g * 16, 16)] = lax.iota(jnp.int32, 16) + g * 16

    _table_pipeline(cid2_hbm, ut_hbm, tailu_hbm, scr_hbm, 0,
                    U_BLOCKS, U_BPW, U_CHUNKS, U_TAIL, wid, rows_v, r128_v,
                    ids_v, pid_v, ppos_v, cid_v, cpos_v,
                    slab_v, rowbuf_v, wpos_v, tail_v, sem)

    _table_pipeline(oid2_hbm, ot_hbm, tailo_hbm, scr_hbm, BATCH,
                    O_BLOCKS, O_BPW, O_CHUNKS, O_TAIL, wid, rows_v, r128_v,
                    ids_v, pid_v, ppos_v, cid_v, cpos_v,
                    slab_v, rowbuf_v, wpos_v, tail_v, sem)


B_PER_W = BATCH // NW


def _tc_merge_body(u_ref, o_ref, out_ref):
    out_ref[...] = jnp.concatenate([u_ref[:, :D], o_ref[:, :D]], axis=1)


_tc_merge = pl.pallas_call(
    _tc_merge_body,
    grid=(NW,),
    in_specs=[
        pl.BlockSpec((B_PER_W, 128), lambda i: (i, 0)),
        pl.BlockSpec((B_PER_W, 128), lambda i: (i + NW, 0)),
    ],
    out_specs=pl.BlockSpec((B_PER_W, 2 * D), lambda i: (i, 0)),
    out_shape=jax.ShapeDtypeStruct((BATCH, 2 * D), jnp.float32),
)


def kernel(clientId, organization, user_table, org_table):
    cid2 = clientId.astype(jnp.int32).reshape(128, 128)
    oid2 = organization.astype(jnp.int32).reshape(128, 128)
    ut = user_table.T
    ot = org_table.T
    tail_u = jnp.pad(user_table[U_TAIL:],
                     ((0, 128 - (U_COLS - U_TAIL)), (0, 96)))
    tail_o = jnp.pad(org_table[O_TAIL:],
                     ((0, 128 - (O_COLS - O_TAIL)), (0, 96)))
    scr = _sweep(cid2, oid2, ut, ot, tail_u, tail_o)
    return _tc_merge(scr, scr)
